# Initial kernel scaffold; baseline (speedup 1.0000x reference)
import jax
import jax.numpy as jnp
from jax.experimental import pallas as pl


def kernel(input, mempool):
    raise NotImplementedError("write your pallas kernel here")


# speedup vs baseline: 16.3322x; 16.3322x over previous
import jax
import jax.numpy as jnp
from jax import lax
from jax.experimental import pallas as pl
from jax.experimental.pallas import tpu as pltpu

_DIM = 512
_NUM_ITEM = 4096
_K = 16
_TR = 256


def _tile_body(x_ref, mp_ref, out1_ref, out2_ref, att_s, cur_s):
    qc = x_ref[0]
    mp = mp_ref[...]
    att = lax.dot_general(
        qc, mp, (((0,), (1,)), ((), ())), preferred_element_type=jnp.float32
    )
    att_s[...] = att
    cur_s[...] = att
    m0 = jnp.max(att, axis=1, keepdims=True)

    def step(_, carry):
        denom, _t = carry
        cur = cur_s[...]
        m = jnp.max(cur, axis=1, keepdims=True)
        denom = denom + jnp.exp(m - m0)
        cur_s[...] = jnp.where(cur == m, -jnp.inf, cur)
        return denom, m

    denom, t = lax.fori_loop(
        0, _K, step, (jnp.zeros((_TR, 1), jnp.float32), m0)
    )
    att = att_s[...]
    p = jnp.where(att >= t, jnp.exp(att - m0), 0.0)
    recip = 1.0 / denom
    out2_ref[...] = p * att * recip
    out1t = lax.dot_general(
        mp, p, (((0,), (1,)), ((), ())), preferred_element_type=jnp.float32
    )
    out1_ref[0] = out1t * jnp.reshape(recip, (1, _TR))


def kernel(input, mempool):
    B, C, H, W = input.shape
    x3 = input.reshape(B, C, H * W)
    rows = B * H * W
    ntiles = rows // _TR
    tpb = (H * W) // _TR
    out1, out2 = pl.pallas_call(
        _tile_body,
        grid=(ntiles,),
        in_specs=[
            pl.BlockSpec((1, C, _TR), lambda i: (i // tpb, 0, i % tpb)),
            pl.BlockSpec((_NUM_ITEM, C), lambda i: (0, 0)),
        ],
        out_specs=[
            pl.BlockSpec((1, C, _TR), lambda i: (i // tpb, 0, i % tpb)),
            pl.BlockSpec((_TR, _NUM_ITEM), lambda i: (i, 0)),
        ],
        out_shape=[
            jax.ShapeDtypeStruct((B, C, H * W), jnp.float32),
            jax.ShapeDtypeStruct((rows, _NUM_ITEM), jnp.float32),
        ],
        scratch_shapes=[
            pltpu.VMEM((_TR, _NUM_ITEM), jnp.float32),
            pltpu.VMEM((_TR, _NUM_ITEM), jnp.float32),
        ],
        compiler_params=pltpu.CompilerParams(
            dimension_semantics=("arbitrary",),
        ),
    )(x3, mempool)
    return out1.reshape(B, C, H, W), out2


# speedup vs baseline: 18.9765x; 1.1619x over previous
import jax
import jax.numpy as jnp
from jax import lax
from jax.experimental import pallas as pl
from jax.experimental.pallas import tpu as pltpu

_DIM = 512
_NUM_ITEM = 4096
_K = 16
_TR = 256


def _tile_body(x_ref, mp_ref, mpb_ref, out1_ref, out2_ref, att_s):
    qc = x_ref[0]
    mp = mp_ref[...]
    att = lax.dot_general(
        qc, mp, (((0,), (1,)), ((), ())), preferred_element_type=jnp.float32
    )
    att_s[...] = att
    m0 = jnp.max(att, axis=1, keepdims=True)

    def step(_, carry):
        denom, m = carry
        cur = att_s[...]
        mn = jnp.max(
            jnp.where(cur < m, cur, -jnp.inf), axis=1, keepdims=True
        )
        return denom + jnp.exp(mn - m0), mn

    denom, t = lax.fori_loop(
        0, _K - 1, step, (jnp.ones((_TR, 1), jnp.float32), m0)
    )
    att = att_s[...]
    p = jnp.where(att >= t, jnp.exp(att - m0), 0.0)
    recip = 1.0 / denom
    out2_ref[...] = p * att * recip
    out1t = lax.dot_general(
        mpb_ref[...],
        p.astype(jnp.bfloat16),
        (((0,), (1,)), ((), ())),
        preferred_element_type=jnp.float32,
    )
    out1_ref[0] = out1t * jnp.reshape(recip, (1, _TR))


def kernel(input, mempool):
    B, C, H, W = input.shape
    x3 = input.reshape(B, C, H * W)
    rows = B * H * W
    ntiles = rows // _TR
    tpb = (H * W) // _TR
    out1, out2 = pl.pallas_call(
        _tile_body,
        grid=(ntiles,),
        in_specs=[
            pl.BlockSpec((1, C, _TR), lambda i: (i // tpb, 0, i % tpb)),
            pl.BlockSpec((_NUM_ITEM, C), lambda i: (0, 0)),
            pl.BlockSpec((_NUM_ITEM, C), lambda i: (0, 0)),
        ],
        out_specs=[
            pl.BlockSpec((1, C, _TR), lambda i: (i // tpb, 0, i % tpb)),
            pl.BlockSpec((_TR, _NUM_ITEM), lambda i: (i, 0)),
        ],
        out_shape=[
            jax.ShapeDtypeStruct((B, C, H * W), jnp.float32),
            jax.ShapeDtypeStruct((rows, _NUM_ITEM), jnp.float32),
        ],
        scratch_shapes=[
            pltpu.VMEM((_TR, _NUM_ITEM), jnp.float32),
        ],
        compiler_params=pltpu.CompilerParams(
            dimension_semantics=("arbitrary",),
        ),
    )(x3, mempool, mempool.astype(jnp.bfloat16))
    return out1.reshape(B, C, H, W), out2


# speedup vs baseline: 20.2413x; 1.0667x over previous
import jax
import jax.numpy as jnp
from jax import lax
from jax.experimental import pallas as pl
from jax.experimental.pallas import tpu as pltpu

_DIM = 512
_NUM_ITEM = 4096
_K = 16
_TR = 256


def _tile_body(x_ref, mp_ref, mpb_ref, out1_ref, out2_ref, att_s):
    qc = x_ref[0]
    mp = mp_ref[...]
    att = lax.dot_general(
        qc, mp, (((0,), (1,)), ((), ())), preferred_element_type=jnp.float32
    )
    att_s[...] = att
    m0 = jnp.max(att, axis=1, keepdims=True)

    def step(_, carry):
        denom, m = carry
        cur = att_s[...]
        ma = jnp.max(
            jnp.where(cur < m, cur, -jnp.inf), axis=1, keepdims=True
        )
        mb = jnp.max(
            jnp.where(cur < ma, cur, -jnp.inf), axis=1, keepdims=True
        )
        return denom + jnp.exp(ma - m0) + jnp.exp(mb - m0), mb

    denom, m14 = lax.fori_loop(
        0, (_K - 2) // 2, step, (jnp.ones((_TR, 1), jnp.float32), m0)
    )
    cur = att_s[...]
    t = jnp.max(jnp.where(cur < m14, cur, -jnp.inf), axis=1, keepdims=True)
    denom = denom + jnp.exp(t - m0)
    att = att_s[...]
    p = jnp.where(att >= t, jnp.exp(att - m0), 0.0)
    recip = 1.0 / denom
    out2_ref[...] = p * att * recip
    out1t = lax.dot_general(
        mpb_ref[...],
        p.astype(jnp.bfloat16),
        (((0,), (1,)), ((), ())),
        preferred_element_type=jnp.float32,
    )
    out1_ref[0] = out1t * jnp.reshape(recip, (1, _TR))


def kernel(input, mempool):
    B, C, H, W = input.shape
    x3 = input.reshape(B, C, H * W)
    rows = B * H * W
    ntiles = rows // _TR
    tpb = (H * W) // _TR
    out1, out2 = pl.pallas_call(
        _tile_body,
        grid=(ntiles,),
        in_specs=[
            pl.BlockSpec((1, C, _TR), lambda i: (i // tpb, 0, i % tpb)),
            pl.BlockSpec((_NUM_ITEM, C), lambda i: (0, 0)),
            pl.BlockSpec((_NUM_ITEM, C), lambda i: (0, 0)),
        ],
        out_specs=[
            pl.BlockSpec((1, C, _TR), lambda i: (i // tpb, 0, i % tpb)),
            pl.BlockSpec((_TR, _NUM_ITEM), lambda i: (i, 0)),
        ],
        out_shape=[
            jax.ShapeDtypeStruct((B, C, H * W), jnp.float32),
            jax.ShapeDtypeStruct((rows, _NUM_ITEM), jnp.float32),
        ],
        scratch_shapes=[
            pltpu.VMEM((_TR, _NUM_ITEM), jnp.float32),
        ],
        compiler_params=pltpu.CompilerParams(
            dimension_semantics=("arbitrary",),
        ),
    )(x3, mempool, mempool.astype(jnp.bfloat16))
    return out1.reshape(B, C, H, W), out2


# speedup vs baseline: 21.5550x; 1.0649x over previous
import jax
import jax.numpy as jnp
from jax import lax
from jax.experimental import pallas as pl
from jax.experimental.pallas import tpu as pltpu

_DIM = 512
_NUM_ITEM = 4096
_K = 16
_TR = 256


def _tile_body(x_ref, mp_ref, mpb_ref, out1_ref, out2_ref, att_s):
    qc = x_ref[0]
    mp = mp_ref[...]
    att = lax.dot_general(
        qc, mp, (((0,), (1,)), ((), ())), preferred_element_type=jnp.float32
    )
    att_s[...] = att
    m0 = jnp.max(att, axis=1, keepdims=True)

    denom = jnp.ones((_TR, 1), jnp.float32)
    m = m0
    cur = att_s[...]
    for _ in range(_K - 1):
        m = jnp.max(
            jnp.where(cur < m, cur, -jnp.inf), axis=1, keepdims=True
        )
        denom = denom + jnp.exp(m - m0)
    t = m
    att = att_s[...]
    p = jnp.where(att >= t, jnp.exp(att - m0), 0.0)
    recip = 1.0 / denom
    out2_ref[...] = p * att * recip
    out1t = lax.dot_general(
        mpb_ref[...],
        p.astype(jnp.bfloat16),
        (((0,), (1,)), ((), ())),
        preferred_element_type=jnp.float32,
    )
    out1_ref[0] = out1t * jnp.reshape(recip, (1, _TR))


def kernel(input, mempool):
    B, C, H, W = input.shape
    x3 = input.reshape(B, C, H * W)
    rows = B * H * W
    ntiles = rows // _TR
    tpb = (H * W) // _TR
    out1, out2 = pl.pallas_call(
        _tile_body,
        grid=(ntiles,),
        in_specs=[
            pl.BlockSpec((1, C, _TR), lambda i: (i // tpb, 0, i % tpb)),
            pl.BlockSpec((_NUM_ITEM, C), lambda i: (0, 0)),
            pl.BlockSpec((_NUM_ITEM, C), lambda i: (0, 0)),
        ],
        out_specs=[
            pl.BlockSpec((1, C, _TR), lambda i: (i // tpb, 0, i % tpb)),
            pl.BlockSpec((_TR, _NUM_ITEM), lambda i: (i, 0)),
        ],
        out_shape=[
            jax.ShapeDtypeStruct((B, C, H * W), jnp.float32),
            jax.ShapeDtypeStruct((rows, _NUM_ITEM), jnp.float32),
        ],
        scratch_shapes=[
            pltpu.VMEM((_TR, _NUM_ITEM), jnp.float32),
        ],
        compiler_params=pltpu.CompilerParams(
            dimension_semantics=("arbitrary",),
        ),
    )(x3, mempool, mempool.astype(jnp.bfloat16))
    return out1.reshape(B, C, H, W), out2
